# Initial kernel scaffold; baseline (speedup 1.0000x reference)
import jax
import jax.numpy as jnp
from jax.experimental import pallas as pl


def kernel(a_tokens, b_tokens, neg_tokens, table):
    raise NotImplementedError("write your pallas kernel here")


# speedup vs baseline: 2.2375x; 2.2375x over previous
import functools

import jax
import jax.numpy as jnp
from jax import lax
from jax.experimental import pallas as pl
from jax.experimental.pallas import tpu as pltpu
from jax.experimental.pallas import tpu_sc as plsc

_NC, _NS = 2, 16
_NW = _NC * _NS


@functools.cache
def _build(B, L, K, V, D):
    R = B * (2 + K)
    CROWS = 5
    CL = CROWS * L
    RPW = R // _NW
    CHUNKS = RPW // CROWS
    assert R % _NW == 0 and RPW % CROWS == 0
    scale = 1.0 / L

    mesh = plsc.VectorSubcoreMesh(core_axis_name="c", subcore_axis_name="s")

    @functools.partial(
        pl.kernel,
        out_type=jax.ShapeDtypeStruct((_NW, RPW, D), jnp.float32),
        mesh=mesh,
        compiler_params=pltpu.CompilerParams(use_tc_tiling_on_sc=False),
        scratch_types=[
            pltpu.VMEM((CHUNKS, CL), jnp.int32),
            pltpu.VMEM((RPW, D), jnp.float32),
            pltpu.VMEM((CL, D), jnp.float32),
            pltpu.SemaphoreType.DMA,
        ],
    )
    def k(table_hbm, idx_hbm, out_hbm, idx_v, out_v, rows_v, sem):
        wid = lax.axis_index("s") * _NC + lax.axis_index("c")
        pltpu.sync_copy(idx_hbm.at[wid], idx_v)

        def body(g, _):
            pltpu.async_copy(table_hbm.at[idx_v.at[g]], rows_v, sem).wait()
            for c in range(CROWS):
                acc = rows_v[c * L]
                for t in range(1, L):
                    acc = acc + rows_v[c * L + t]
                out_v[g * CROWS + c] = acc * scale
            return _

        lax.fori_loop(0, CHUNKS, body, 0)
        pltpu.sync_copy(out_v, out_hbm.at[wid])

    return k, CHUNKS, CL


def kernel(a_tokens, b_tokens, neg_tokens, table):
    B, L = a_tokens.shape
    K = neg_tokens.shape[1]
    V, D = table.shape
    k, CHUNKS, CL = _build(B, L, K, V, D)

    tokens = jnp.concatenate(
        [a_tokens[:, None, :], b_tokens[:, None, :], neg_tokens], axis=1
    ).astype(jnp.int32)
    idx = tokens.reshape(_NW, CHUNKS, CL)

    out = k(table, idx)
    out = out.reshape(B, 2 + K, D)
    return (out[:, 0:1, :], out[:, 1:2, :], out[:, 2:, :])
